# Initial kernel scaffold; baseline (speedup 1.0000x reference)
import jax
import jax.numpy as jnp
from jax.experimental import pallas as pl


def kernel(x, route_feature, task_id, params):
    raise NotImplementedError("write your pallas kernel here")


# speedup vs baseline: 3.4843x; 3.4843x over previous
import functools

import jax
import jax.numpy as jnp
from jax.experimental import pallas as pl

E = 8


def _moe_body(tok_ref, lnw_ref, lnb_ref, wr_ref, rfeat_ref, wt_ref, temb_ref,
              w1_ref, b1_ref, w2_ref, b2_ref, y_ref, me_ref, fe_ref,
              *, tiles_per_image, d):
    i = pl.program_id(0)
    tok = tok_ref[...]
    tt = tok.shape[0]

    mu = jnp.mean(tok, axis=1, keepdims=True)
    cen = tok - mu
    var = jnp.mean(cen * cen, axis=1, keepdims=True)
    t = cen * jax.lax.rsqrt(var + 1e-6) * lnw_ref[...] + lnb_ref[...]

    rf = jnp.dot(rfeat_ref[...], wt_ref[...], preferred_element_type=jnp.float32)
    b_idx = i // tiles_per_image
    bsel = (jax.lax.broadcasted_iota(jnp.int32, (rf.shape[0], 1), 0) == b_idx)
    bias = jnp.sum(rf * bsel.astype(jnp.float32), axis=0, keepdims=True) + temb_ref[...]
    logits = jnp.dot(t, wr_ref[...], preferred_element_type=jnp.float32) + bias

    m = jnp.max(logits, axis=1, keepdims=True)
    ex = jnp.exp(logits - m)
    probs = ex / jnp.sum(ex, axis=1, keepdims=True)

    eidx = jax.lax.broadcasted_iota(jnp.int32, (tt, E), 1)
    v1 = m
    i1 = jnp.min(jnp.where(logits == v1, eidx, E), axis=1, keepdims=True)
    masked = jnp.where(eidx == i1, -jnp.inf, logits)
    v2 = jnp.max(masked, axis=1, keepdims=True)
    i2 = jnp.min(jnp.where(masked == v2, eidx, E), axis=1, keepdims=True)
    r = jnp.exp(v2 - v1)
    g1 = 1.0 / (1.0 + r)
    g2 = r / (1.0 + r)
    is1 = (eidx == i1).astype(jnp.float32)
    is2 = (eidx == i2).astype(jnp.float32)
    combine = is1 * g1 + is2 * g2
    dispatch = is1 + is2

    @pl.when(i == 0)
    def _():
        me_ref[...] = jnp.zeros_like(me_ref)
        fe_ref[...] = jnp.zeros_like(fe_ref)

    me_ref[...] += jnp.sum(probs, axis=0, keepdims=True)
    fe_ref[...] += jnp.sum(dispatch, axis=0, keepdims=True)

    h = jax.nn.gelu(jnp.dot(t, w1_ref[...], preferred_element_type=jnp.float32)
                    + b1_ref[...])
    cexp = jnp.broadcast_to(combine[:, :, None], (tt, E, 2 * d)).reshape(tt, 2 * E * d)
    out = jnp.dot(h * cexp, w2_ref[...], preferred_element_type=jnp.float32)
    out = out + jnp.dot(combine, b2_ref[...], preferred_element_type=jnp.float32)
    y_ref[...] = tok + out


def _moe_block(tokens, route_feature, temb, p, tile, tiles_per_image):
    T, d = tokens.shape
    w1cat = p['W1'].transpose(1, 0, 2).reshape(d, 2 * E * d)
    b1cat = p['b1'].reshape(1, 2 * E * d)
    w2cat = p['W2'].reshape(2 * E * d, d)
    grid = T // tile
    full = lambda a: pl.BlockSpec(a.shape, lambda i: (0,) * a.ndim)
    y, me, fe = pl.pallas_call(
        functools.partial(_moe_body, tiles_per_image=tiles_per_image, d=d),
        grid=(grid,),
        in_specs=[
            pl.BlockSpec((tile, d), lambda i: (i, 0)),
            full(p['ln_w'].reshape(1, d)),
            full(p['ln_b'].reshape(1, d)),
            full(p['Wr']),
            full(route_feature),
            full(p['Wt']),
            full(temb.reshape(1, E)),
            full(w1cat),
            full(b1cat),
            full(w2cat),
            full(p['b2']),
        ],
        out_specs=[
            pl.BlockSpec((tile, d), lambda i: (i, 0)),
            pl.BlockSpec((1, E), lambda i: (0, 0)),
            pl.BlockSpec((1, E), lambda i: (0, 0)),
        ],
        out_shape=[
            jax.ShapeDtypeStruct((T, d), jnp.float32),
            jax.ShapeDtypeStruct((1, E), jnp.float32),
            jax.ShapeDtypeStruct((1, E), jnp.float32),
        ],
    )(tokens, p['ln_w'].reshape(1, d), p['ln_b'].reshape(1, d), p['Wr'],
      route_feature, p['Wt'], temb.reshape(1, E), w1cat, b1cat, w2cat, p['b2'])
    me = me[0] / T
    fe = fe[0] / T
    std = E * jnp.sum(me * fe)
    mi = jnp.sum(me * jnp.log(me + 1e-9))
    return y, std, mi


def _conv2d(x, w, b=None):
    out = jax.lax.conv_general_dilated(x, w, (1, 1), [(1, 1), (1, 1)],
                                       dimension_numbers=('NCHW', 'OIHW', 'NCHW'))
    if b is not None:
        out = out + b[None, :, None, None]
    return out


def _pixel_unshuffle(x, r=2):
    B, C, H, W = x.shape
    x = x.reshape(B, C, H // r, r, W // r, r)
    x = x.transpose(0, 1, 3, 5, 2, 4)
    return x.reshape(B, C * r * r, H // r, W // r)


def _stage(x_img, route_feature, temb, p, target_tile):
    B, C, H, W = x_img.shape
    tokens = x_img.transpose(0, 2, 3, 1).reshape(-1, C)
    tile = (H * W) // max(1, (H * W) // target_tile)
    tiles_per_image = (H * W) // tile
    y, std, mi = _moe_block(tokens, route_feature, temb, p, tile, tiles_per_image)
    y_img = y.reshape(B, H, W, C).transpose(0, 3, 1, 2)
    return y_img, std, mi


def kernel(x, route_feature, task_id, params):
    p = params
    x1 = _conv2d(x, p['conv0_w'], p['conv0_b'])

    temb = lambda blk: jnp.take(p[blk]['task_emb'], task_id, axis=0)

    y1, s1, m1 = _stage(x1, route_feature, temb('blk1'), p['blk1'], 3136)
    x2 = _pixel_unshuffle(_conv2d(y1, p['down1_w']))
    y2, s2, m2 = _stage(x2, route_feature, temb('blk2'), p['blk2'], 1568)
    x3 = _pixel_unshuffle(_conv2d(y2, p['down2_w']))
    y3, s3, m3 = _stage(x3, route_feature, temb('blk3'), p['blk3'], 1568)
    x4 = _pixel_unshuffle(_conv2d(y3, p['down3_w']))
    y4, s4, m4 = _stage(x4, route_feature, temb('blk4'), p['blk4'], 784)
    x5 = _pixel_unshuffle(_conv2d(y4, p['down4_w']))

    std = s1 + s2 + s3 + s4
    mi = m1 + m2 + m3 + m4
    return x1, x2, x3, x4, x5, std, mi


# speedup vs baseline: 4.3255x; 1.2414x over previous
import functools

import jax
import jax.numpy as jnp
from jax.experimental import pallas as pl

E = 8


def _moe_body(tok_ref, lnw_ref, lnb_ref, wr_ref, rfeat_ref, wt_ref, temb_ref,
              w1_ref, b1_ref, w2_ref, b2_ref, sel_ref, y_ref, me_ref, fe_ref,
              *, tiles_per_image, d):
    i = pl.program_id(0)
    tok = tok_ref[...]
    tt = tok.shape[0]

    mu = jnp.mean(tok, axis=1, keepdims=True)
    cen = tok - mu
    var = jnp.mean(cen * cen, axis=1, keepdims=True)
    t = cen * jax.lax.rsqrt(var + 1e-6) * lnw_ref[...] + lnb_ref[...]

    rf = jnp.dot(rfeat_ref[...], wt_ref[...], preferred_element_type=jnp.float32)
    b_idx = i // tiles_per_image
    bsel = (jax.lax.broadcasted_iota(jnp.int32, (rf.shape[0], 1), 0) == b_idx)
    bias = jnp.sum(rf * bsel.astype(jnp.float32), axis=0, keepdims=True) + temb_ref[...]
    logits = jnp.dot(t, wr_ref[...], preferred_element_type=jnp.float32) + bias

    m = jnp.max(logits, axis=1, keepdims=True)
    ex = jnp.exp(logits - m)
    probs = ex / jnp.sum(ex, axis=1, keepdims=True)

    eidx = jax.lax.broadcasted_iota(jnp.int32, (tt, E), 1)
    v1 = m
    i1 = jnp.min(jnp.where(logits == v1, eidx, E), axis=1, keepdims=True)
    masked = jnp.where(eidx == i1, -jnp.inf, logits)
    v2 = jnp.max(masked, axis=1, keepdims=True)
    i2 = jnp.min(jnp.where(masked == v2, eidx, E), axis=1, keepdims=True)
    r = jnp.exp(v2 - v1)
    g1 = 1.0 / (1.0 + r)
    g2 = r / (1.0 + r)
    is1 = (eidx == i1).astype(jnp.float32)
    is2 = (eidx == i2).astype(jnp.float32)
    combine = is1 * g1 + is2 * g2
    dispatch = is1 + is2

    @pl.when(i == 0)
    def _():
        me_ref[...] = jnp.zeros_like(me_ref)
        fe_ref[...] = jnp.zeros_like(fe_ref)

    me_ref[...] += jnp.sum(probs, axis=0, keepdims=True)
    fe_ref[...] += jnp.sum(dispatch, axis=0, keepdims=True)

    h = jax.nn.gelu(jnp.dot(t, w1_ref[...], preferred_element_type=jnp.float32)
                    + b1_ref[...])
    cexp = jnp.dot(combine, sel_ref[...], preferred_element_type=jnp.float32)
    out = jnp.dot(h * cexp, w2_ref[...], preferred_element_type=jnp.float32)
    out = out + jnp.dot(combine, b2_ref[...], preferred_element_type=jnp.float32)
    y_ref[...] = tok + out


def _moe_block(tokens, route_feature, temb, p, tile, tiles_per_image):
    T, d = tokens.shape
    w1cat = p['W1'].transpose(1, 0, 2).reshape(d, 2 * E * d)
    b1cat = p['b1'].reshape(1, 2 * E * d)
    w2cat = p['W2'].reshape(2 * E * d, d)
    sel = (jnp.arange(2 * E * d)[None, :] // (2 * d) ==
           jnp.arange(E)[:, None]).astype(jnp.float32)
    grid = T // tile
    full = lambda a: pl.BlockSpec(a.shape, lambda i: (0,) * a.ndim)
    y, me, fe = pl.pallas_call(
        functools.partial(_moe_body, tiles_per_image=tiles_per_image, d=d),
        grid=(grid,),
        in_specs=[
            pl.BlockSpec((tile, d), lambda i: (i, 0)),
            full(p['ln_w'].reshape(1, d)),
            full(p['ln_b'].reshape(1, d)),
            full(p['Wr']),
            full(route_feature),
            full(p['Wt']),
            full(temb.reshape(1, E)),
            full(w1cat),
            full(b1cat),
            full(w2cat),
            full(p['b2']),
            full(sel),
        ],
        out_specs=[
            pl.BlockSpec((tile, d), lambda i: (i, 0)),
            pl.BlockSpec((1, E), lambda i: (0, 0)),
            pl.BlockSpec((1, E), lambda i: (0, 0)),
        ],
        out_shape=[
            jax.ShapeDtypeStruct((T, d), jnp.float32),
            jax.ShapeDtypeStruct((1, E), jnp.float32),
            jax.ShapeDtypeStruct((1, E), jnp.float32),
        ],
    )(tokens, p['ln_w'].reshape(1, d), p['ln_b'].reshape(1, d), p['Wr'],
      route_feature, p['Wt'], temb.reshape(1, E), w1cat, b1cat, w2cat, p['b2'],
      sel)
    me = me[0] / T
    fe = fe[0] / T
    std = E * jnp.sum(me * fe)
    mi = jnp.sum(me * jnp.log(me + 1e-9))
    return y, std, mi


def _conv2d(x, w, b=None):
    out = jax.lax.conv_general_dilated(x, w, (1, 1), [(1, 1), (1, 1)],
                                       dimension_numbers=('NCHW', 'OIHW', 'NCHW'))
    if b is not None:
        out = out + b[None, :, None, None]
    return out


def _pixel_unshuffle(x, r=2):
    B, C, H, W = x.shape
    x = x.reshape(B, C, H // r, r, W // r, r)
    x = x.transpose(0, 1, 3, 5, 2, 4)
    return x.reshape(B, C * r * r, H // r, W // r)


def _stage(x_img, route_feature, temb, p, target_tile):
    B, C, H, W = x_img.shape
    tokens = x_img.transpose(0, 2, 3, 1).reshape(-1, C)
    tile = (H * W) // max(1, (H * W) // target_tile)
    tiles_per_image = (H * W) // tile
    y, std, mi = _moe_block(tokens, route_feature, temb, p, tile, tiles_per_image)
    y_img = y.reshape(B, H, W, C).transpose(0, 3, 1, 2)
    return y_img, std, mi


def kernel(x, route_feature, task_id, params):
    p = params
    x1 = _conv2d(x, p['conv0_w'], p['conv0_b'])

    temb = lambda blk: jnp.take(p[blk]['task_emb'], task_id, axis=0)

    y1, s1, m1 = _stage(x1, route_feature, temb('blk1'), p['blk1'], 3136)
    x2 = _pixel_unshuffle(_conv2d(y1, p['down1_w']))
    y2, s2, m2 = _stage(x2, route_feature, temb('blk2'), p['blk2'], 1568)
    x3 = _pixel_unshuffle(_conv2d(y2, p['down2_w']))
    y3, s3, m3 = _stage(x3, route_feature, temb('blk3'), p['blk3'], 1568)
    x4 = _pixel_unshuffle(_conv2d(y3, p['down3_w']))
    y4, s4, m4 = _stage(x4, route_feature, temb('blk4'), p['blk4'], 784)
    x5 = _pixel_unshuffle(_conv2d(y4, p['down4_w']))

    std = s1 + s2 + s3 + s4
    mi = m1 + m2 + m3 + m4
    return x1, x2, x3, x4, x5, std, mi
